# Initial kernel scaffold; baseline (speedup 1.0000x reference)
import jax
import jax.numpy as jnp
from jax.experimental import pallas as pl


def kernel(anchors, deltas, scores):
    raise NotImplementedError("write your pallas kernel here")


# speedup vs baseline: 6.1144x; 6.1144x over previous
import math

import jax
import jax.numpy as jnp
from jax.experimental import pallas as pl
from jax.experimental.pallas import tpu as pltpu

_N = 20000
_NPAD = 20480
_NCHUNK = 20
_K = 1000
_NMS_THRESH = 0.7
_IMG_H = 800.0
_IMG_W = 800.0
_SCALE_CLAMP = math.log(1000.0 / 16.0)
_NEG = -1e9
_NINF = float("-inf")


def _flat_iota():
    s = jax.lax.broadcasted_iota(jnp.int32, (8, 128), 0)
    l = jax.lax.broadcasted_iota(jnp.int32, (8, 128), 1)
    return s * 128 + l


def _rpn_body(x1a, y1a, x2a, y2a, dx, dy, dw, dh, sc,
              ox1, oy1, ox2, oy2, osc,
              px1, py1, px2, py2, pms):
    ax1 = x1a[...]
    ay1 = y1a[...]
    ax2 = x2a[...]
    ay2 = y2a[...]
    w = ax2 - ax1
    h = ay2 - ay1
    cx = ax1 + 0.5 * w
    cy = ay1 + 0.5 * h
    dwc = jnp.minimum(dw[...], _SCALE_CLAMP)
    dhc = jnp.minimum(dh[...], _SCALE_CLAMP)
    pcx = dx[...] * w + cx
    pcy = dy[...] * h + cy
    pw = jnp.exp(dwc) * w
    ph = jnp.exp(dhc) * h
    bx1 = jnp.clip(pcx - 0.5 * pw, 0.0, _IMG_W)
    by1 = jnp.clip(pcy - 0.5 * ph, 0.0, _IMG_H)
    bx2 = jnp.clip(pcx + 0.5 * pw, 0.0, _IMG_W)
    by2 = jnp.clip(pcy + 0.5 * ph, 0.0, _IMG_H)
    valid = ((bx2 - bx1) > 0.0) & ((by2 - by1) > 0.0)
    ms = jnp.where(valid, sc[...], _NEG)
    px1[...] = bx1
    py1[...] = by1
    px2[...] = bx2
    py2[...] = by2
    pms[...] = ms

    iota2 = _flat_iota()
    zeros = jnp.zeros((8, 128), jnp.float32)

    def initcm_body(c, cm):
        return jnp.where(iota2 == c, jnp.max(pms[pl.ds(c, 1)]), cm)

    cm0 = jax.lax.fori_loop(0, _NCHUNK, initcm_body,
                            jnp.full((8, 128), _NINF))

    def extract_body(i, carry):
        cm, tx1, ty1, tx2, ty2, ts = carry
        m = jnp.max(cm)
        c = jnp.min(jnp.where(cm == m, iota2, jnp.int32(10 ** 9)))
        chunk = pms[pl.ds(c, 1)][0]
        li = jnp.min(jnp.where(chunk == m, iota2, jnp.int32(10 ** 9)))
        oh = iota2 == li
        x1i = jnp.sum(jnp.where(oh, px1[pl.ds(c, 1)][0], zeros))
        y1i = jnp.sum(jnp.where(oh, py1[pl.ds(c, 1)][0], zeros))
        x2i = jnp.sum(jnp.where(oh, px2[pl.ds(c, 1)][0], zeros))
        y2i = jnp.sum(jnp.where(oh, py2[pl.ds(c, 1)][0], zeros))
        newchunk = jnp.where(oh, _NINF, chunk)
        pms[pl.ds(c, 1)] = newchunk[None]
        cm = jnp.where(iota2 == c, jnp.max(newchunk), cm)
        slot = iota2 == i
        tx1 = jnp.where(slot, x1i, tx1)
        ty1 = jnp.where(slot, y1i, ty1)
        tx2 = jnp.where(slot, x2i, tx2)
        ty2 = jnp.where(slot, y2i, ty2)
        ts = jnp.where(slot, m, ts)
        return cm, tx1, ty1, tx2, ty2, ts

    init = (cm0, zeros, zeros, zeros, zeros, jnp.full((8, 128), _NEG))
    _, tx1, ty1, tx2, ty2, ts = jax.lax.fori_loop(0, _K, extract_body, init)

    ta = (tx2 - tx1) * (ty2 - ty1)

    def nms_body(i, carry):
        keep, kcnt, rx1, ry1, rx2, ry2, rs = carry
        ohi = iota2 == i
        x1i = jnp.sum(jnp.where(ohi, tx1, zeros))
        y1i = jnp.sum(jnp.where(ohi, ty1, zeros))
        x2i = jnp.sum(jnp.where(ohi, tx2, zeros))
        y2i = jnp.sum(jnp.where(ohi, ty2, zeros))
        si = jnp.sum(jnp.where(ohi, ts, zeros))
        ai = jnp.sum(jnp.where(ohi, ta, zeros))
        alive = jnp.sum(jnp.where(ohi, keep, zeros)) > 0.5
        iw = jnp.maximum(jnp.minimum(x2i, tx2) - jnp.maximum(x1i, tx1), 0.0)
        ih = jnp.maximum(jnp.minimum(y2i, ty2) - jnp.maximum(y1i, ty1), 0.0)
        inter = iw * ih
        union = ai + ta - inter
        iou = jnp.where(union > 0.0,
                        inter / jnp.maximum(union, 1e-9), 0.0)
        sup = (iou > _NMS_THRESH) & (iota2 > i) & alive
        keep = jnp.where(sup, 0.0, keep)
        g = alive & (si > _NEG * 0.5)
        oslot = (iota2 == kcnt) & g
        rx1 = jnp.where(oslot, x1i, rx1)
        ry1 = jnp.where(oslot, y1i, ry1)
        rx2 = jnp.where(oslot, x2i, rx2)
        ry2 = jnp.where(oslot, y2i, ry2)
        rs = jnp.where(oslot, si, rs)
        kcnt = kcnt + jnp.where(g, 1, 0).astype(jnp.int32)
        return keep, kcnt, rx1, ry1, rx2, ry2, rs

    init_c = (jnp.ones((8, 128), jnp.float32), jnp.int32(0),
              zeros, zeros, zeros, zeros, zeros)
    _, _, rx1, ry1, rx2, ry2, rs = jax.lax.fori_loop(0, _K, nms_body, init_c)

    ox1[...] = rx1
    oy1[...] = ry1
    ox2[...] = rx2
    oy2[...] = ry2
    osc[...] = rs


def kernel(anchors, deltas, scores):
    at = jnp.pad(anchors, ((0, _NPAD - _N), (0, 0))).T
    dt = jnp.pad(deltas, ((0, _NPAD - _N), (0, 0))).T
    sp = jnp.pad(scores, (0, _NPAD - _N))
    planes = [a.reshape(_NCHUNK, 8, 128) for a in at] + \
             [d.reshape(_NCHUNK, 8, 128) for d in dt] + \
             [sp.reshape(_NCHUNK, 8, 128)]
    out = pl.pallas_call(
        _rpn_body,
        out_shape=[jax.ShapeDtypeStruct((8, 128), jnp.float32)] * 5,
        scratch_shapes=[pltpu.VMEM((_NCHUNK, 8, 128), jnp.float32)] * 5,
    )(*planes)
    cols = [o.reshape(_NPAD // _NCHUNK)[:_K] for o in out]
    return jnp.stack(cols, axis=-1)


# speedup vs baseline: 6.2156x; 1.0166x over previous
import math

import jax
import jax.numpy as jnp
from jax.experimental import pallas as pl
from jax.experimental.pallas import tpu as pltpu

_N = 20000
_NPAD = 20480
_NCHUNK = 20
_K = 1000
_NMS_THRESH = 0.7
_IMG_H = 800.0
_IMG_W = 800.0
_SCALE_CLAMP = math.log(1000.0 / 16.0)
_NEG = -1e9
_NINF = float("-inf")


def _flat_iota():
    s = jax.lax.broadcasted_iota(jnp.int32, (8, 128), 0)
    l = jax.lax.broadcasted_iota(jnp.int32, (8, 128), 1)
    return s * 128 + l


def _rpn_body(x1a, y1a, x2a, y2a, dx, dy, dw, dh, sc,
              ox1, oy1, ox2, oy2, osc,
              px1, py1, px2, py2, pms,
              smx1, smy1, smx2, smy2, sms):
    ax1 = x1a[...]
    ay1 = y1a[...]
    ax2 = x2a[...]
    ay2 = y2a[...]
    w = ax2 - ax1
    h = ay2 - ay1
    cx = ax1 + 0.5 * w
    cy = ay1 + 0.5 * h
    dwc = jnp.minimum(dw[...], _SCALE_CLAMP)
    dhc = jnp.minimum(dh[...], _SCALE_CLAMP)
    pcx = dx[...] * w + cx
    pcy = dy[...] * h + cy
    pw = jnp.exp(dwc) * w
    ph = jnp.exp(dhc) * h
    bx1 = jnp.clip(pcx - 0.5 * pw, 0.0, _IMG_W)
    by1 = jnp.clip(pcy - 0.5 * ph, 0.0, _IMG_H)
    bx2 = jnp.clip(pcx + 0.5 * pw, 0.0, _IMG_W)
    by2 = jnp.clip(pcy + 0.5 * ph, 0.0, _IMG_H)
    valid = ((bx2 - bx1) > 0.0) & ((by2 - by1) > 0.0)
    ms = jnp.where(valid, sc[...], _NEG)
    px1[...] = bx1
    py1[...] = by1
    px2[...] = bx2
    py2[...] = by2
    pms[...] = ms

    iota2 = _flat_iota()
    zeros = jnp.zeros((8, 128), jnp.float32)

    def initcm_body(c, cm):
        return jnp.where(iota2 == c, jnp.max(pms[pl.ds(c, 1)]), cm)

    cm0 = jax.lax.fori_loop(0, _NCHUNK, initcm_body,
                            jnp.full((8, 128), _NINF))

    def extract_body(i, carry):
        cm, tx1, ty1, tx2, ty2, ts = carry
        m = jnp.max(cm)
        c = jnp.min(jnp.where(cm == m, iota2, jnp.int32(10 ** 9)))
        chunk = pms[pl.ds(c, 1)][0]
        li = jnp.min(jnp.where(chunk == m, iota2, jnp.int32(10 ** 9)))
        oh = iota2 == li
        x1i = jnp.sum(jnp.where(oh, px1[pl.ds(c, 1)][0], zeros))
        y1i = jnp.sum(jnp.where(oh, py1[pl.ds(c, 1)][0], zeros))
        x2i = jnp.sum(jnp.where(oh, px2[pl.ds(c, 1)][0], zeros))
        y2i = jnp.sum(jnp.where(oh, py2[pl.ds(c, 1)][0], zeros))
        newchunk = jnp.where(oh, _NINF, chunk)
        pms[pl.ds(c, 1)] = newchunk[None]
        cm = jnp.where(iota2 == c, jnp.max(newchunk), cm)
        smx1[i] = x1i
        smy1[i] = y1i
        smx2[i] = x2i
        smy2[i] = y2i
        sms[i] = m
        slot = iota2 == i
        tx1 = jnp.where(slot, x1i, tx1)
        ty1 = jnp.where(slot, y1i, ty1)
        tx2 = jnp.where(slot, x2i, tx2)
        ty2 = jnp.where(slot, y2i, ty2)
        ts = jnp.where(slot, m, ts)
        return cm, tx1, ty1, tx2, ty2, ts

    init = (cm0, zeros, zeros, zeros, zeros, jnp.full((8, 128), _NEG))
    _, tx1, ty1, tx2, ty2, ts = jax.lax.fori_loop(0, _K, extract_body, init)

    ta = (tx2 - tx1) * (ty2 - ty1)

    def nms_body(i, carry):
        keep, kcnt, rx1, ry1, rx2, ry2, rs = carry
        ohi = iota2 == i
        x1i = smx1[i]
        y1i = smy1[i]
        x2i = smx2[i]
        y2i = smy2[i]
        si = sms[i]
        ai = (x2i - x1i) * (y2i - y1i)
        alive = jnp.sum(jnp.where(ohi, keep, zeros)) > 0.5
        iw = jnp.maximum(jnp.minimum(x2i, tx2) - jnp.maximum(x1i, tx1), 0.0)
        ih = jnp.maximum(jnp.minimum(y2i, ty2) - jnp.maximum(y1i, ty1), 0.0)
        inter = iw * ih
        union = ai + ta - inter
        iou = jnp.where(union > 0.0,
                        inter / jnp.maximum(union, 1e-9), 0.0)
        sup = (iou > _NMS_THRESH) & (iota2 > i) & alive
        keep = jnp.where(sup, 0.0, keep)
        g = alive & (si > _NEG * 0.5)
        oslot = (iota2 == kcnt) & g
        rx1 = jnp.where(oslot, x1i, rx1)
        ry1 = jnp.where(oslot, y1i, ry1)
        rx2 = jnp.where(oslot, x2i, rx2)
        ry2 = jnp.where(oslot, y2i, ry2)
        rs = jnp.where(oslot, si, rs)
        kcnt = kcnt + jnp.where(g, 1, 0).astype(jnp.int32)
        return keep, kcnt, rx1, ry1, rx2, ry2, rs

    init_c = (jnp.ones((8, 128), jnp.float32), jnp.int32(0),
              zeros, zeros, zeros, zeros, zeros)
    _, _, rx1, ry1, rx2, ry2, rs = jax.lax.fori_loop(0, _K, nms_body, init_c)

    ox1[...] = rx1
    oy1[...] = ry1
    ox2[...] = rx2
    oy2[...] = ry2
    osc[...] = rs


def kernel(anchors, deltas, scores):
    at = jnp.pad(anchors, ((0, _NPAD - _N), (0, 0))).T
    dt = jnp.pad(deltas, ((0, _NPAD - _N), (0, 0))).T
    sp = jnp.pad(scores, (0, _NPAD - _N))
    planes = [a.reshape(_NCHUNK, 8, 128) for a in at] + \
             [d.reshape(_NCHUNK, 8, 128) for d in dt] + \
             [sp.reshape(_NCHUNK, 8, 128)]
    out = pl.pallas_call(
        _rpn_body,
        out_shape=[jax.ShapeDtypeStruct((8, 128), jnp.float32)] * 5,
        scratch_shapes=[pltpu.VMEM((_NCHUNK, 8, 128), jnp.float32)] * 5
        + [pltpu.SMEM((_K,), jnp.float32)] * 5,
    )(*planes)
    cols = [o.reshape(_NPAD // _NCHUNK)[:_K] for o in out]
    return jnp.stack(cols, axis=-1)
